# Initial kernel scaffold; baseline (speedup 1.0000x reference)
import jax
import jax.numpy as jnp
from jax.experimental import pallas as pl


def kernel(text_idx, edge_src, edge_dst, etypes, emb, Wm, bm, W_ih, W_hh, b_ih, b_hh, W1, b1, W2, b2):
    raise NotImplementedError("write your pallas kernel here")


# speedup vs baseline: 24.6165x; 24.6165x over previous
import functools

import jax
import jax.numpy as jnp
from jax import lax
from jax.experimental import pallas as pl
from jax.experimental.pallas import tpu as pltpu
from jax.experimental.pallas import tpu_sc as plsc

N = 10000
E = 320000
D = 128
K = 3
L = 2
STEPS = 2

NC = 2
NS = 16
NW = NC * NS

CHUNK = 80
EPW = E // NW
NCH = EPW // CHUNK
NODE_CH = N // CHUNK
EMB_CPW = -(-NODE_CH // NW)
RPT = 624
RPT_LAST = N - (NS - 1) * RPT

BN = 1000

_SC_MESH = plsc.VectorSubcoreMesh(core_axis_name="c", subcore_axis_name="s")


@functools.partial(
    pl.kernel,
    mesh=_SC_MESH,
    out_type=jax.ShapeDtypeStruct((N, D), jnp.float32),
    scratch_types=[
        pltpu.VMEM((1, CHUNK), jnp.int32),
        pltpu.VMEM((CHUNK, D), jnp.float32),
        pltpu.SemaphoreType.DMA,
    ],
)
def _embed_gather_k(emb_hbm, idx_hbm, out_hbm, idx_v, rows_v, sem):
    c = lax.axis_index("c")
    s = lax.axis_index("s")
    w = s * NC + c

    def body(i, carry):
        cid = w * EMB_CPW + i

        @pl.when(cid < NODE_CH)
        def _():
            pltpu.sync_copy(idx_hbm.at[cid], idx_v)
            pltpu.async_copy(emb_hbm.at[idx_v.at[0]], rows_v, sem).wait()
            base = pl.multiple_of(cid * CHUNK, 8)
            pltpu.sync_copy(rows_v, out_hbm.at[pl.ds(base, CHUNK)])

        return carry

    lax.fori_loop(0, EMB_CPW, body, 0)


@functools.partial(
    pl.kernel,
    mesh=_SC_MESH,
    out_type=jax.ShapeDtypeStruct((NC, N, D), jnp.float32),
    scratch_types=[
        pltpu.VMEM((NCH, CHUNK), jnp.int32),
        pltpu.VMEM((NCH, CHUNK), jnp.int32),
        pltpu.VMEM((CHUNK, D), jnp.float32),
        pltpu.VMEM_SHARED((N, D), jnp.float32),
        pltpu.SemaphoreType.DMA,
    ],
)
def _segsum_k(proj_hbm, src_hbm, dst_hbm, zeros_hbm, out_hbm,
              src_v, dst_v, rows_v, acc_sh, sem):
    c = lax.axis_index("c")
    s = lax.axis_index("s")
    w = s * NC + c

    base_r = pl.multiple_of(s * RPT, 8)

    @pl.when(s < NS - 1)
    def _():
        pltpu.sync_copy(zeros_hbm.at[pl.ds(base_r, RPT)],
                        acc_sh.at[pl.ds(base_r, RPT)])

    @pl.when(s == NS - 1)
    def _():
        pltpu.sync_copy(zeros_hbm.at[pl.ds((NS - 1) * RPT, RPT_LAST)],
                        acc_sh.at[pl.ds((NS - 1) * RPT, RPT_LAST)])

    pltpu.sync_copy(src_hbm.at[w], src_v)
    pltpu.sync_copy(dst_hbm.at[w], dst_v)
    plsc.subcore_barrier()

    def body(i, carry):
        pltpu.async_copy(proj_hbm.at[src_v.at[i]], rows_v, sem).wait()
        pltpu.sync_copy(rows_v, acc_sh.at[dst_v.at[i]], add=True)
        return carry

    lax.fori_loop(0, NCH, body, 0)
    plsc.subcore_barrier()

    @pl.when(s < NS - 1)
    def _():
        pltpu.sync_copy(acc_sh.at[pl.ds(base_r, RPT)],
                        out_hbm.at[c, pl.ds(base_r, RPT)])

    @pl.when(s == NS - 1)
    def _():
        pltpu.sync_copy(acc_sh.at[pl.ds((NS - 1) * RPT, RPT_LAST)],
                        out_hbm.at[c, pl.ds((NS - 1) * RPT, RPT_LAST)])


def _proj_body(h_ref, wt_ref, b_ref, out_ref):
    out_ref[0] = (
        jnp.dot(h_ref[...], wt_ref[0], preferred_element_type=jnp.float32)
        + b_ref[0]
    )


def _proj(h, wmt, bm3):
    return pl.pallas_call(
        _proj_body,
        grid=(K, N // BN),
        in_specs=[
            pl.BlockSpec((BN, D), lambda k, n: (n, 0)),
            pl.BlockSpec((1, D, D), lambda k, n: (k, 0, 0)),
            pl.BlockSpec((1, 1, D), lambda k, n: (k, 0, 0)),
        ],
        out_specs=pl.BlockSpec((1, BN, D), lambda k, n: (k, n, 0)),
        out_shape=jax.ShapeDtypeStruct((K, N, D), jnp.float32),
    )(h, wmt, bm3)


def _gru_body(acc_ref, h_ref, wih_ref, whh_ref, bih_ref, bhh_ref,
              out_ref, sum_ref):
    a = acc_ref[0] + acc_ref[1]
    h = h_ref[...]
    gi = jnp.dot(a, wih_ref[...], preferred_element_type=jnp.float32) + bih_ref[...]
    gh = jnp.dot(h, whh_ref[...], preferred_element_type=jnp.float32) + bhh_ref[...]
    r = jax.nn.sigmoid(gi[:, :D] + gh[:, :D])
    z = jax.nn.sigmoid(gi[:, D:2 * D] + gh[:, D:2 * D])
    n = jnp.tanh(gi[:, 2 * D:] + r * gh[:, 2 * D:])
    hn = (1.0 - z) * n + z * h
    out_ref[...] = hn
    part = jnp.sum(hn, axis=0, keepdims=True)
    i = pl.program_id(0)

    @pl.when(i == 0)
    def _():
        sum_ref[...] = part

    @pl.when(i != 0)
    def _():
        sum_ref[...] += part

    @pl.when(i == pl.num_programs(0) - 1)
    def _():
        sum_ref[...] *= (1.0 / N)


def _gru(acc2, h, wih_t, whh_t, bih2, bhh2):
    return pl.pallas_call(
        _gru_body,
        grid=(N // BN,),
        in_specs=[
            pl.BlockSpec((NC, BN, D), lambda n: (0, n, 0)),
            pl.BlockSpec((BN, D), lambda n: (n, 0)),
            pl.BlockSpec((D, 3 * D), lambda n: (0, 0)),
            pl.BlockSpec((D, 3 * D), lambda n: (0, 0)),
            pl.BlockSpec((1, 3 * D), lambda n: (0, 0)),
            pl.BlockSpec((1, 3 * D), lambda n: (0, 0)),
        ],
        out_specs=[
            pl.BlockSpec((BN, D), lambda n: (n, 0)),
            pl.BlockSpec((1, D), lambda n: (0, 0)),
        ],
        out_shape=[
            jax.ShapeDtypeStruct((N, D), jnp.float32),
            jax.ShapeDtypeStruct((1, D), jnp.float32),
        ],
    )(acc2, h, wih_t, whh_t, bih2, bhh2)


def _mean_body(h_ref, sum_ref):
    part = jnp.sum(h_ref[...], axis=0, keepdims=True)
    i = pl.program_id(0)

    @pl.when(i == 0)
    def _():
        sum_ref[...] = part

    @pl.when(i != 0)
    def _():
        sum_ref[...] += part

    @pl.when(i == pl.num_programs(0) - 1)
    def _():
        sum_ref[...] *= (1.0 / N)


def _colmean(h):
    return pl.pallas_call(
        _mean_body,
        grid=(N // BN,),
        in_specs=[pl.BlockSpec((BN, D), lambda n: (n, 0))],
        out_specs=pl.BlockSpec((1, D), lambda n: (0, 0)),
        out_shape=jax.ShapeDtypeStruct((1, D), jnp.float32),
    )(h)


def _head_body(agg_ref, w1t_ref, b1_ref, w2_ref, b2_ref, res_ref):
    hidden = jnp.dot(agg_ref[...], w1t_ref[...],
                     preferred_element_type=jnp.float32) + b1_ref[...]
    hidden = jnp.maximum(hidden, 0.0)
    res_ref[...] = jnp.sum(hidden * w2_ref[...], axis=1, keepdims=True) + b2_ref[...]


def _head(agg, w1t, b1r, w2, b2r):
    return pl.pallas_call(
        _head_body,
        in_specs=[
            pl.BlockSpec(agg.shape, lambda: (0, 0)),
            pl.BlockSpec(w1t.shape, lambda: (0, 0)),
            pl.BlockSpec(b1r.shape, lambda: (0, 0)),
            pl.BlockSpec(w2.shape, lambda: (0, 0)),
            pl.BlockSpec(b2r.shape, lambda: (0, 0)),
        ],
        out_specs=pl.BlockSpec((1, 1), lambda: (0, 0)),
        out_shape=jax.ShapeDtypeStruct((1, 1), jnp.float32),
    )(agg, w1t, b1r, w2, b2r)


def kernel(text_idx, edge_src, edge_dst, etypes, emb, Wm, bm,
           W_ih, W_hh, b_ih, b_hh, W1, b1, W2, b2):
    idx2d = text_idx.astype(jnp.int32).reshape(NODE_CH, 1, CHUNK)
    flat_src = (etypes.astype(jnp.int32) * N + edge_src.astype(jnp.int32))
    src2d = flat_src.reshape(NW, NCH, CHUNK)
    dst2d = edge_dst.astype(jnp.int32).reshape(NW, NCH, CHUNK)
    zeros_nd = jnp.zeros((N, D), jnp.float32)

    h = _embed_gather_k(emb, idx2d)
    means = [_colmean(h)]
    for l in range(L):
        wmt = jnp.transpose(Wm[l], (0, 2, 1))
        bm3 = bm[l].reshape(K, 1, D)
        wih_t = W_ih[l].T
        whh_t = W_hh[l].T
        bih2 = b_ih[l].reshape(1, 3 * D)
        bhh2 = b_hh[l].reshape(1, 3 * D)
        colmean = None
        for _ in range(STEPS):
            proj = _proj(h, wmt, bm3)
            acc2 = _segsum_k(proj.reshape(K * N, D), src2d, dst2d, zeros_nd)
            h, colmean = _gru(acc2, h, wih_t, whh_t, bih2, bhh2)
        means.append(colmean)
    agg = jnp.concatenate(means, axis=1)
    res = _head(agg, W1.T, b1.reshape(1, D), W2, b2.reshape(1, 1))
    return (res, agg)


# speedup vs baseline: 37.0527x; 1.5052x over previous
import functools

import jax
import jax.numpy as jnp
from jax import lax
from jax.experimental import pallas as pl
from jax.experimental.pallas import tpu as pltpu
from jax.experimental.pallas import tpu_sc as plsc

N = 10000
E = 320000
D = 128
K = 3
L = 2
STEPS = 2

NC = 2
NS = 16
NW = NC * NS

CHUNK = 80
EPW = E // NW
NCH = EPW // CHUNK
PH = 64
ECH = 80
NODE_CH = N // ECH
EMB_CPW = -(-NODE_CH // NW)
RPT = 624
RPT_LAST = N - (NS - 1) * RPT

BN = 1000

_SC_MESH = plsc.VectorSubcoreMesh(core_axis_name="c", subcore_axis_name="s")


@functools.partial(
    pl.kernel,
    mesh=_SC_MESH,
    out_type=jax.ShapeDtypeStruct((N, D), jnp.float32),
    scratch_types=[
        pltpu.VMEM((1, ECH), jnp.int32),
        pltpu.VMEM((ECH, D), jnp.float32),
        pltpu.SemaphoreType.DMA,
    ],
)
def _embed_gather_k(emb_hbm, idx_hbm, out_hbm, idx_v, rows_v, sem):
    c = lax.axis_index("c")
    s = lax.axis_index("s")
    w = s * NC + c

    def body(i, carry):
        cid = w * EMB_CPW + i

        @pl.when(cid < NODE_CH)
        def _():
            pltpu.sync_copy(idx_hbm.at[cid], idx_v)
            pltpu.async_copy(emb_hbm.at[idx_v.at[0]], rows_v, sem).wait()
            base = pl.multiple_of(cid * ECH, 8)
            pltpu.sync_copy(rows_v, out_hbm.at[pl.ds(base, ECH)])

        return carry

    lax.fori_loop(0, EMB_CPW, body, 0)


@functools.partial(
    pl.kernel,
    mesh=_SC_MESH,
    out_type=jax.ShapeDtypeStruct((NC, N, D), jnp.float32),
    scratch_types=[
        pltpu.VMEM((PH, CHUNK), jnp.int32),
        pltpu.VMEM((PH, CHUNK), jnp.int32),
        pltpu.VMEM((CHUNK, D), jnp.float32),
        pltpu.VMEM((CHUNK, D), jnp.float32),
        pltpu.VMEM_SHARED((N, D), jnp.float32),
        pltpu.SemaphoreType.DMA,
        pltpu.SemaphoreType.DMA,
    ],
)
def _segsum_k(proj_hbm, src_hbm, dst_hbm, zeros_hbm, out_hbm,
              src_v, dst_v, rows_a, rows_b, acc_sh, sem_a, sem_b):
    c = lax.axis_index("c")
    s = lax.axis_index("s")
    w = s * NC + c

    base_r = pl.multiple_of(s * RPT, 8)

    @pl.when(s < NS - 1)
    def _():
        pltpu.sync_copy(zeros_hbm.at[pl.ds(base_r, RPT)],
                        acc_sh.at[pl.ds(base_r, RPT)])

    @pl.when(s == NS - 1)
    def _():
        pltpu.sync_copy(zeros_hbm.at[pl.ds((NS - 1) * RPT, RPT_LAST)],
                        acc_sh.at[pl.ds((NS - 1) * RPT, RPT_LAST)])

    plsc.subcore_barrier()

    for start, cnt in ((0, PH), (PH, NCH - PH)):
        pltpu.sync_copy(src_hbm.at[w, pl.ds(start, cnt)],
                        src_v.at[pl.ds(0, cnt)])
        pltpu.sync_copy(dst_hbm.at[w, pl.ds(start, cnt)],
                        dst_v.at[pl.ds(0, cnt)])
        pltpu.async_copy(proj_hbm.at[src_v.at[0]], rows_a, sem_a)
        pltpu.async_copy(proj_hbm.at[src_v.at[1]], rows_b, sem_b)

        def body(i, carry):
            pltpu.make_async_copy(proj_hbm.at[src_v.at[2 * i]], rows_a,
                                  sem_a).wait()
            pltpu.sync_copy(rows_a, acc_sh.at[dst_v.at[2 * i]], add=True)

            @pl.when(2 * i + 2 < cnt)
            def _():
                pltpu.async_copy(proj_hbm.at[src_v.at[2 * i + 2]], rows_a,
                                 sem_a)

            pltpu.make_async_copy(proj_hbm.at[src_v.at[2 * i + 1]], rows_b,
                                  sem_b).wait()
            pltpu.sync_copy(rows_b, acc_sh.at[dst_v.at[2 * i + 1]], add=True)

            @pl.when(2 * i + 3 < cnt)
            def _():
                pltpu.async_copy(proj_hbm.at[src_v.at[2 * i + 3]], rows_b,
                                 sem_b)

            return carry

        lax.fori_loop(0, cnt // 2, body, 0)
        if cnt % 2:
            pltpu.make_async_copy(proj_hbm.at[src_v.at[cnt - 1]], rows_a,
                                  sem_a).wait()
            pltpu.sync_copy(rows_a, acc_sh.at[dst_v.at[cnt - 1]], add=True)

    plsc.subcore_barrier()

    @pl.when(s < NS - 1)
    def _():
        pltpu.sync_copy(acc_sh.at[pl.ds(base_r, RPT)],
                        out_hbm.at[c, pl.ds(base_r, RPT)])

    @pl.when(s == NS - 1)
    def _():
        pltpu.sync_copy(acc_sh.at[pl.ds((NS - 1) * RPT, RPT_LAST)],
                        out_hbm.at[c, pl.ds((NS - 1) * RPT, RPT_LAST)])


def _proj_body(h_ref, wt_ref, b_ref, out_ref):
    out_ref[0] = (
        jnp.dot(h_ref[...], wt_ref[0], preferred_element_type=jnp.float32)
        + b_ref[0]
    )


def _proj(h, wmt, bm3):
    return pl.pallas_call(
        _proj_body,
        grid=(K, N // BN),
        in_specs=[
            pl.BlockSpec((BN, D), lambda k, n: (n, 0)),
            pl.BlockSpec((1, D, D), lambda k, n: (k, 0, 0)),
            pl.BlockSpec((1, 1, D), lambda k, n: (k, 0, 0)),
        ],
        out_specs=pl.BlockSpec((1, BN, D), lambda k, n: (k, n, 0)),
        out_shape=jax.ShapeDtypeStruct((K, N, D), jnp.float32),
    )(h, wmt, bm3)


def _gru_body(acc_ref, h_ref, wih_ref, whh_ref, bih_ref, bhh_ref,
              out_ref, sum_ref):
    a = acc_ref[0] + acc_ref[1]
    h = h_ref[...]
    gi = jnp.dot(a, wih_ref[...], preferred_element_type=jnp.float32) + bih_ref[...]
    gh = jnp.dot(h, whh_ref[...], preferred_element_type=jnp.float32) + bhh_ref[...]
    r = jax.nn.sigmoid(gi[:, :D] + gh[:, :D])
    z = jax.nn.sigmoid(gi[:, D:2 * D] + gh[:, D:2 * D])
    n = jnp.tanh(gi[:, 2 * D:] + r * gh[:, 2 * D:])
    hn = (1.0 - z) * n + z * h
    out_ref[...] = hn
    part = jnp.sum(hn, axis=0, keepdims=True)
    i = pl.program_id(0)

    @pl.when(i == 0)
    def _():
        sum_ref[...] = part

    @pl.when(i != 0)
    def _():
        sum_ref[...] += part

    @pl.when(i == pl.num_programs(0) - 1)
    def _():
        sum_ref[...] *= (1.0 / N)


def _gru(acc2, h, wih_t, whh_t, bih2, bhh2):
    return pl.pallas_call(
        _gru_body,
        grid=(N // BN,),
        in_specs=[
            pl.BlockSpec((NC, BN, D), lambda n: (0, n, 0)),
            pl.BlockSpec((BN, D), lambda n: (n, 0)),
            pl.BlockSpec((D, 3 * D), lambda n: (0, 0)),
            pl.BlockSpec((D, 3 * D), lambda n: (0, 0)),
            pl.BlockSpec((1, 3 * D), lambda n: (0, 0)),
            pl.BlockSpec((1, 3 * D), lambda n: (0, 0)),
        ],
        out_specs=[
            pl.BlockSpec((BN, D), lambda n: (n, 0)),
            pl.BlockSpec((1, D), lambda n: (0, 0)),
        ],
        out_shape=[
            jax.ShapeDtypeStruct((N, D), jnp.float32),
            jax.ShapeDtypeStruct((1, D), jnp.float32),
        ],
    )(acc2, h, wih_t, whh_t, bih2, bhh2)


def _mean_body(h_ref, sum_ref):
    part = jnp.sum(h_ref[...], axis=0, keepdims=True)
    i = pl.program_id(0)

    @pl.when(i == 0)
    def _():
        sum_ref[...] = part

    @pl.when(i != 0)
    def _():
        sum_ref[...] += part

    @pl.when(i == pl.num_programs(0) - 1)
    def _():
        sum_ref[...] *= (1.0 / N)


def _colmean(h):
    return pl.pallas_call(
        _mean_body,
        grid=(N // BN,),
        in_specs=[pl.BlockSpec((BN, D), lambda n: (n, 0))],
        out_specs=pl.BlockSpec((1, D), lambda n: (0, 0)),
        out_shape=jax.ShapeDtypeStruct((1, D), jnp.float32),
    )(h)


def _head_body(agg_ref, w1t_ref, b1_ref, w2_ref, b2_ref, res_ref):
    hidden = jnp.dot(agg_ref[...], w1t_ref[...],
                     preferred_element_type=jnp.float32) + b1_ref[...]
    hidden = jnp.maximum(hidden, 0.0)
    res_ref[...] = jnp.sum(hidden * w2_ref[...], axis=1, keepdims=True) + b2_ref[...]


def _head(agg, w1t, b1r, w2, b2r):
    return pl.pallas_call(
        _head_body,
        in_specs=[
            pl.BlockSpec(agg.shape, lambda: (0, 0)),
            pl.BlockSpec(w1t.shape, lambda: (0, 0)),
            pl.BlockSpec(b1r.shape, lambda: (0, 0)),
            pl.BlockSpec(w2.shape, lambda: (0, 0)),
            pl.BlockSpec(b2r.shape, lambda: (0, 0)),
        ],
        out_specs=pl.BlockSpec((1, 1), lambda: (0, 0)),
        out_shape=jax.ShapeDtypeStruct((1, 1), jnp.float32),
    )(agg, w1t, b1r, w2, b2r)


def kernel(text_idx, edge_src, edge_dst, etypes, emb, Wm, bm,
           W_ih, W_hh, b_ih, b_hh, W1, b1, W2, b2):
    idx2d = text_idx.astype(jnp.int32).reshape(NODE_CH, 1, ECH)
    flat_src = (etypes.astype(jnp.int32) * N + edge_src.astype(jnp.int32))
    src2d = flat_src.reshape(NW, NCH, CHUNK)
    dst2d = edge_dst.astype(jnp.int32).reshape(NW, NCH, CHUNK)
    zeros_nd = jnp.zeros((N, D), jnp.float32)

    h = _embed_gather_k(emb, idx2d)
    means = [_colmean(h)]
    for l in range(L):
        wmt = jnp.transpose(Wm[l], (0, 2, 1))
        bm3 = bm[l].reshape(K, 1, D)
        wih_t = W_ih[l].T
        whh_t = W_hh[l].T
        bih2 = b_ih[l].reshape(1, 3 * D)
        bhh2 = b_hh[l].reshape(1, 3 * D)
        colmean = None
        for _ in range(STEPS):
            proj = _proj(h, wmt, bm3)
            acc2 = _segsum_k(proj.reshape(K * N, D), src2d, dst2d, zeros_nd)
            h, colmean = _gru(acc2, h, wih_t, whh_t, bih2, bhh2)
        means.append(colmean)
    agg = jnp.concatenate(means, axis=1)
    res = _head(agg, W1.T, b1.reshape(1, D), W2, b2.reshape(1, 1))
    return (res, agg)
